# Initial kernel scaffold; baseline (speedup 1.0000x reference)
import jax
import jax.numpy as jnp
from jax.experimental import pallas as pl


def kernel(token, cond, W, b):
    raise NotImplementedError("write your pallas kernel here")


# speedup vs baseline: 1.6204x; 1.6204x over previous
import jax
import jax.numpy as jnp
from jax.experimental import pallas as pl
from jax.experimental.pallas import tpu as pltpu

_TAU = 5.0
_K = 4096
_TL = 1024


def _logits_body(tok_ref, cond_ref, w_ref, b_ref, out_ref):
    t = tok_ref[...] + cond_ref[0]
    tb = t.astype(jnp.bfloat16).astype(jnp.float32)
    wb = w_ref[...].astype(jnp.bfloat16).astype(jnp.float32)
    out_ref[...] = jnp.sum(tb * wb, axis=1, keepdims=True) + b_ref[0]


def _mask_body(logits_ref, g_ref, mask_ref):
    s = (logits_ref[...] + g_ref[...]) / _TAU
    m = jnp.max(s, axis=1, keepdims=True)
    y = jnp.exp(s - m)
    denom = jnp.sum(y, axis=1, keepdims=True)
    q = y / denom
    u = jax.lax.bitcast_convert_type(q, jnp.int32)
    B = u.shape[0]

    def vstep(i, v):
        bit = 30 - i
        trial = v | (1 << bit)
        cnt = jnp.sum((u >= trial).astype(jnp.int32), axis=1, keepdims=True)
        return jnp.where(cnt >= _K, trial, v)

    vk = jax.lax.fori_loop(0, 31, vstep, jnp.zeros((B, 1), jnp.int32))

    gt = u > vk
    eq = u == vk
    need = _K - jnp.sum(gt.astype(jnp.int32), axis=1, keepdims=True)
    idx = jax.lax.broadcasted_iota(jnp.int32, u.shape, 1)

    def mstep(i, mm):
        bit = 13 - i
        trial = mm | (1 << bit)
        c = jnp.sum((eq & (idx < trial)).astype(jnp.int32), axis=1, keepdims=True)
        return jnp.where(c <= need, trial, mm)

    mend = jax.lax.fori_loop(0, 14, mstep, jnp.zeros((B, 1), jnp.int32))
    mask_ref[...] = (gt | (eq & (idx < mend))).astype(jnp.float32)


def kernel(token, cond, W, b):
    B, L, D = token.shape
    g = jax.random.gumbel(jax.random.key(42), (B, L), jnp.float32)

    tok2 = token.reshape(B * L, D)
    cond3 = cond.reshape(B, 1, D)
    blocks_per_batch = L // _TL
    logits = pl.pallas_call(
        _logits_body,
        grid=(B * L // _TL,),
        in_specs=[
            pl.BlockSpec((_TL, D), lambda j: (j, 0)),
            pl.BlockSpec((1, 1, D), lambda j: (j // blocks_per_batch, 0, 0)),
            pl.BlockSpec((1, D), lambda j: (0, 0)),
            pl.BlockSpec(memory_space=pltpu.SMEM),
        ],
        out_specs=pl.BlockSpec((_TL, 1), lambda j: (j, 0)),
        out_shape=jax.ShapeDtypeStruct((B * L, 1), jnp.float32),
    )(tok2, cond3, W, b)
    logits = logits.reshape(B, L)

    mask = pl.pallas_call(
        _mask_body,
        out_shape=jax.ShapeDtypeStruct((B, L), jnp.float32),
    )(logits, g)

    return (mask, logits)
